# baseline (speedup 1.0000x reference)
import jax
import jax.numpy as jnp
from jax import lax
from jax.experimental import pallas as pl
from jax.experimental.pallas import tpu as pltpu
from jax.experimental.pallas import tpu_sc as plsc

_N = 10000
_E = 160000
_DIN = 1152
_H = 128
_B = 32
_C = 10

_NC = 2
_NS = 16
_CHUNK = 128
_TCH = 40
_EROWS = _NC * _NS * _TCH
_R0 = 64
_R1 = 16
_EPAD = _EROWS * _CHUNK
_ROWS_ALLOC = _EROWS + 64
_EPAD2 = _ROWS_ALLOC * _CHUNK
_SLAB = 640
_RPAD = _NS * _SLAB

_BN = 1000
_G = _N // _BN

_f32 = jnp.float32


def _sc_agg_body(src_h, dst_h, y_h, z_h, p_h,
                 src_v, dst_v, rows0_v, rows1_v, acc, sem):
    c = lax.axis_index("c")
    s = lax.axis_index("s")
    slab = pl.ds(s * _SLAB, _SLAB)

    with jax.named_scope("agg_zero"):
        pltpu.sync_copy(z_h, acc.at[slab])

    with jax.named_scope("agg_stage"):
        base = jnp.where(c == 0, s * _R0, _NS * _R0 + s * _R1)
        nch = jnp.where(c == 0, _R0, _R1)
        pltpu.sync_copy(src_h.at[pl.ds(base, _R0)], src_v)
        pltpu.sync_copy(dst_h.at[pl.ds(base, _R0)], dst_v)

    plsc.subcore_barrier()

    def gather(j, buf):
        pltpu.async_copy(y_h.at[src_v.at[j]], buf, sem)

    def drain():
        pltpu.make_async_copy(y_h.at[src_v.at[0]], rows0_v, sem).wait()

    def scat(j, buf):
        pltpu.sync_copy(buf, acc.at[dst_v.at[j]], add=True)

    with jax.named_scope("agg_edges"):
        gather(0, rows0_v)

        def step(i, carry):
            j = 2 * i
            drain()
            gather(j + 1, rows1_v)
            scat(j, rows0_v)
            drain()
            gather(j + 2, rows0_v)
            scat(j + 1, rows1_v)
            return carry

        lax.fori_loop(0, nch // 2 - 1, step, 0)

        j = nch - 2
        drain()
        gather(j + 1, rows1_v)
        scat(j, rows0_v)
        drain()
        scat(j + 1, rows1_v)

    plsc.subcore_barrier()

    with jax.named_scope("agg_wb"):
        pltpu.sync_copy(acc.at[slab], p_h.at[c, slab])


def _make_sc_agg():
    mesh = plsc.VectorSubcoreMesh(
        core_axis_name="c", subcore_axis_name="s",
        num_cores=_NC, num_subcores=_NS)
    out_type = jax.ShapeDtypeStruct((_NC, _RPAD, _H), _f32)
    scratch = [
        pltpu.VMEM((_R0, _CHUNK), jnp.int32),
        pltpu.VMEM((_R0, _CHUNK), jnp.int32),
        pltpu.VMEM((_CHUNK, _H), _f32),
        pltpu.VMEM((_CHUNK, _H), _f32),
        pltpu.VMEM_SHARED((_RPAD, _H), _f32),
        pltpu.SemaphoreType.DMA,
    ]
    return pl.kernel(
        _sc_agg_body, out_type=out_type, mesh=mesh, scratch_types=scratch)


def _sc_cnt_body(dst_h, z_h, ones_h, c_h, dst_v, ones_v, cacc):
    c = lax.axis_index("c")
    s = lax.axis_index("s")
    slab = pl.ds(s * _SLAB, _SLAB)

    pltpu.sync_copy(z_h, cacc.at[slab])
    pltpu.sync_copy(ones_h, ones_v)
    base = (c * _NS + s) * _TCH
    pltpu.sync_copy(dst_h.at[pl.ds(base, _TCH)], dst_v)

    plsc.subcore_barrier()

    def step(j, carry):
        pltpu.sync_copy(ones_v, cacc.at[dst_v.at[j]], add=True)
        return carry

    lax.fori_loop(0, _TCH, step, 0)

    plsc.subcore_barrier()

    pltpu.sync_copy(cacc.at[slab], c_h.at[c, slab])


def _make_sc_cnt():
    mesh = plsc.VectorSubcoreMesh(
        core_axis_name="c", subcore_axis_name="s",
        num_cores=_NC, num_subcores=_NS)
    out_type = jax.ShapeDtypeStruct((_NC, _RPAD, _H), _f32)
    scratch = [
        pltpu.VMEM((_TCH, _CHUNK), jnp.int32),
        pltpu.VMEM((_CHUNK, _H), _f32),
        pltpu.VMEM_SHARED((_RPAD, _H), _f32),
    ]
    return pl.kernel(
        _sc_cnt_body, out_type=out_type, mesh=mesh, scratch_types=scratch)


def _tc_a_body(x_r, wl_r, wr_r, b_r, yl_r, yr_r):
    xb = x_r[...]
    yl_r[...] = jnp.dot(xb, wl_r[...], preferred_element_type=_f32)
    yr_r[...] = jnp.dot(xb, wr_r[...], preferred_element_type=_f32) + b_r[...]


def _tc_a(x, wl, wr, b):
    return pl.pallas_call(
        _tc_a_body,
        grid=(_G,),
        in_specs=[
            pl.BlockSpec((_BN, _DIN), lambda i: (i, 0)),
            pl.BlockSpec((_DIN, _H), lambda i: (0, 0)),
            pl.BlockSpec((_DIN, _H), lambda i: (0, 0)),
            pl.BlockSpec((1, _H), lambda i: (0, 0)),
        ],
        out_specs=[
            pl.BlockSpec((_BN, _H), lambda i: (i, 0)),
            pl.BlockSpec((_BN, _H), lambda i: (i, 0)),
        ],
        out_shape=[jax.ShapeDtypeStruct((_N, _H), _f32)] * 2,
    )(x, wl, wr, b)


def _combine(p0_r, p1_r, c0_r, c1_r, yr_r, relu):
    cnts = (c0_r[...] + c1_r[...]).reshape(_BN, _H)[:, 0:1]
    inv = 1.0 / jnp.maximum(cnts, 1.0)
    h = (p0_r[...] + p1_r[...]).reshape(_BN, _H) * inv + yr_r[...]
    if relu:
        h = jnp.maximum(h, 0.0)
    return h


def _p_specs():
    return [
        pl.BlockSpec((1, _BN, _H), lambda i: (0, i, 0)),
        pl.BlockSpec((1, _BN, _H), lambda i: (1, i, 0)),
        pl.BlockSpec((1, _BN, _H), lambda i: (0, i, 0)),
        pl.BlockSpec((1, _BN, _H), lambda i: (1, i, 0)),
    ]


def _tc_b_body(p0_r, p1_r, c0_r, c1_r, yrp_r, wl_r, wr_r, b_r, yl_r, yr_r):
    h = _combine(p0_r, p1_r, c0_r, c1_r, yrp_r, relu=True)
    yl_r[...] = jnp.dot(h, wl_r[...], preferred_element_type=_f32)
    yr_r[...] = jnp.dot(h, wr_r[...], preferred_element_type=_f32) + b_r[...]


def _tc_b(p, cn, yrp, wl, wr, b):
    return pl.pallas_call(
        _tc_b_body,
        grid=(_G,),
        in_specs=_p_specs() + [
            pl.BlockSpec((_BN, _H), lambda i: (i, 0)),
            pl.BlockSpec((_H, _H), lambda i: (0, 0)),
            pl.BlockSpec((_H, _H), lambda i: (0, 0)),
            pl.BlockSpec((1, _H), lambda i: (0, 0)),
        ],
        out_specs=[
            pl.BlockSpec((_BN, _H), lambda i: (i, 0)),
            pl.BlockSpec((_BN, _H), lambda i: (i, 0)),
        ],
        out_shape=[jax.ShapeDtypeStruct((_N, _H), _f32)] * 2,
    )(p, p, cn, cn, yrp, wl, wr, b)


def _tc_c_body(p0_r, p1_r, c0_r, c1_r, yrp_r, batch_r, wlin_r, blin_r,
               out_r, gsum, gcnt):
    i = pl.program_id(0)

    @pl.when(i == 0)
    def _():
        gsum[...] = jnp.zeros((_B, _H), _f32)
        gcnt[...] = jnp.zeros((_B, _H), _f32)

    h = _combine(p0_r, p1_r, c0_r, c1_r, yrp_r, relu=False)
    gids = batch_r[...].reshape(1, _BN)
    oh_t = (lax.broadcasted_iota(jnp.int32, (_B, 1), 0) == gids
            ).astype(_f32)
    gsum[...] += jnp.dot(oh_t, h, preferred_element_type=_f32)
    gcnt[...] += jnp.dot(oh_t, jnp.ones((_BN, _H), _f32),
                         preferred_element_type=_f32)

    @pl.when(i == _G - 1)
    def _():
        g = gsum[...] / jnp.maximum(gcnt[...], 1.0)
        out_r[...] = jnp.dot(g, wlin_r[...], preferred_element_type=_f32) \
            + blin_r[...]


def _tc_c(p, cn, yrp, batch3, wlin, blin):
    return pl.pallas_call(
        _tc_c_body,
        grid=(_G,),
        in_specs=_p_specs() + [
            pl.BlockSpec((_BN, _H), lambda i: (i, 0)),
            pl.BlockSpec((1, 1, _BN), lambda i: (i, 0, 0)),
            pl.BlockSpec((_H, _H), lambda i: (0, 0)),
            pl.BlockSpec((1, _H), lambda i: (0, 0)),
        ],
        out_specs=pl.BlockSpec((_B, _H), lambda i: (0, 0)),
        out_shape=jax.ShapeDtypeStruct((_B, _H), _f32),
        scratch_shapes=[
            pltpu.VMEM((_B, _H), _f32),
            pltpu.VMEM((_B, _H), _f32),
        ],
    )(p, p, cn, cn, yrp, batch3, wlin, blin)


def kernel(x, edge_index, batch, W1l, W1r, b1, W2l, W2r, b2, W3l, W3r, b3,
           Wlin, blin):
    src = edge_index[0]
    dst = edge_index[1]
    pad = _EPAD2 - _E
    srcp = jnp.concatenate(
        [src, jnp.zeros((pad,), jnp.int32)]).reshape(_ROWS_ALLOC, _CHUNK)
    dstp = jnp.concatenate(
        [dst, jnp.full((pad,), _N, jnp.int32)]).reshape(_ROWS_ALLOC, _CHUNK)

    z_slab = jnp.zeros((_SLAB, _H), _f32)
    ones_blk = jnp.ones((_CHUNK, _H), _f32)

    b1r = b1.reshape(1, _H)
    b2r = b2.reshape(1, _H)
    b3r = b3.reshape(1, _H)
    wlin_p = jnp.pad(Wlin, ((0, 0), (0, _H - _C)))
    blin_p = jnp.pad(blin, (0, _H - _C)).reshape(1, _H)
    batch3 = batch.reshape(_G, 1, _BN)

    agg = _make_sc_agg()
    cntk = _make_sc_cnt()

    yl1, yr1 = _tc_a(x, W1l, W1r, b1r)
    cn = cntk(dstp, z_slab, ones_blk)
    p = agg(srcp, dstp, yl1, z_slab)
    yl2, yr2 = _tc_b(p, cn, yr1, W2l, W2r, b2r)
    p = agg(srcp, dstp, yl2, z_slab)
    yl3, yr3 = _tc_b(p, cn, yr2, W3l, W3r, b3r)
    p = agg(srcp, dstp, yl3, z_slab)
    out = _tc_c(p, cn, yr3, batch3, wlin_p, blin_p)
    return out[:, :_C]


# speedup vs baseline: 2.2256x; 2.2256x over previous
import jax
import jax.numpy as jnp
from jax import lax
from jax.experimental import pallas as pl
from jax.experimental.pallas import tpu as pltpu
from jax.experimental.pallas import tpu_sc as plsc

_N = 10000
_E = 160000
_DIN = 1152
_H = 128
_B = 32
_C = 10

_NC = 2
_NS = 16
_CHUNK = 128
_TCH = 40
_EROWS = _NC * _NS * _TCH
_R0 = 40
_R1 = 40
_EPAD = _EROWS * _CHUNK
_ROWS_ALLOC = _EROWS + 64
_EPAD2 = _ROWS_ALLOC * _CHUNK
_SLAB = 640
_RPAD = _NS * _SLAB

_BN = 1000
_G = _N // _BN

_f32 = jnp.float32


def _sc_agg_body(src_h, dst_h, y_h, z_h, p_h,
                 src_v, dst_v, rows0_v, rows1_v, acc, sem):
    c = lax.axis_index("c")
    s = lax.axis_index("s")
    slab = pl.ds(s * _SLAB, _SLAB)

    with jax.named_scope("agg_zero"):
        pltpu.sync_copy(z_h, acc.at[slab])

    with jax.named_scope("agg_stage"):
        base = jnp.where(c == 0, s * _R0, _NS * _R0 + s * _R1)
        nch = jnp.where(c == 0, _R0, _R1)
        pltpu.sync_copy(src_h.at[pl.ds(base, _R0)], src_v)
        pltpu.sync_copy(dst_h.at[pl.ds(base, _R0)], dst_v)

    plsc.subcore_barrier()

    def gather(j, buf):
        pltpu.async_copy(y_h.at[src_v.at[j]], buf, sem)

    def drain():
        pltpu.make_async_copy(y_h.at[src_v.at[0]], rows0_v, sem).wait()

    def scat(j, buf):
        pltpu.sync_copy(buf, acc.at[dst_v.at[j]], add=True)

    with jax.named_scope("agg_edges"):
        gather(0, rows0_v)

        def step(i, carry):
            j = 2 * i
            drain()
            gather(j + 1, rows1_v)
            scat(j, rows0_v)
            drain()
            gather(j + 2, rows0_v)
            scat(j + 1, rows1_v)
            return carry

        lax.fori_loop(0, nch // 2 - 1, step, 0)

        j = nch - 2
        drain()
        gather(j + 1, rows1_v)
        scat(j, rows0_v)
        drain()
        scat(j + 1, rows1_v)

    plsc.subcore_barrier()

    with jax.named_scope("agg_wb"):
        pltpu.sync_copy(acc.at[slab], p_h.at[c, slab])


def _make_sc_agg():
    mesh = plsc.VectorSubcoreMesh(
        core_axis_name="c", subcore_axis_name="s",
        num_cores=_NC, num_subcores=_NS)
    out_type = jax.ShapeDtypeStruct((_NC, _RPAD, _H), _f32)
    scratch = [
        pltpu.VMEM((_R0, _CHUNK), jnp.int32),
        pltpu.VMEM((_R0, _CHUNK), jnp.int32),
        pltpu.VMEM((_CHUNK, _H), _f32),
        pltpu.VMEM((_CHUNK, _H), _f32),
        pltpu.VMEM_SHARED((_RPAD, _H), _f32),
        pltpu.SemaphoreType.DMA,
    ]
    return pl.kernel(
        _sc_agg_body, out_type=out_type, mesh=mesh, scratch_types=scratch)


def _sc_cnt_body(dst_h, z_h, ones_h, c_h, dst_v, ones_v, cacc):
    c = lax.axis_index("c")
    s = lax.axis_index("s")
    slab = pl.ds(s * _SLAB, _SLAB)

    pltpu.sync_copy(z_h, cacc.at[slab])
    pltpu.sync_copy(ones_h, ones_v)
    base = (c * _NS + s) * _TCH
    pltpu.sync_copy(dst_h.at[pl.ds(base, _TCH)], dst_v)

    plsc.subcore_barrier()

    def step(j, carry):
        pltpu.sync_copy(ones_v, cacc.at[dst_v.at[j]], add=True)
        return carry

    lax.fori_loop(0, _TCH, step, 0)

    plsc.subcore_barrier()

    pltpu.sync_copy(cacc.at[slab], c_h.at[c, slab])


def _make_sc_cnt():
    mesh = plsc.VectorSubcoreMesh(
        core_axis_name="c", subcore_axis_name="s",
        num_cores=_NC, num_subcores=_NS)
    out_type = jax.ShapeDtypeStruct((_NC, _RPAD, _H), _f32)
    scratch = [
        pltpu.VMEM((_TCH, _CHUNK), jnp.int32),
        pltpu.VMEM((_CHUNK, _H), _f32),
        pltpu.VMEM_SHARED((_RPAD, _H), _f32),
    ]
    return pl.kernel(
        _sc_cnt_body, out_type=out_type, mesh=mesh, scratch_types=scratch)


def _tc_a_body(x_r, wl_r, wr_r, b_r, yl_r, yr_r):
    xb = x_r[...]
    yl_r[...] = jnp.dot(xb, wl_r[...], preferred_element_type=_f32)
    yr_r[...] = jnp.dot(xb, wr_r[...], preferred_element_type=_f32) + b_r[...]


def _tc_a(x, wl, wr, b):
    return pl.pallas_call(
        _tc_a_body,
        grid=(_G,),
        in_specs=[
            pl.BlockSpec((_BN, _DIN), lambda i: (i, 0)),
            pl.BlockSpec((_DIN, _H), lambda i: (0, 0)),
            pl.BlockSpec((_DIN, _H), lambda i: (0, 0)),
            pl.BlockSpec((1, _H), lambda i: (0, 0)),
        ],
        out_specs=[
            pl.BlockSpec((_BN, _H), lambda i: (i, 0)),
            pl.BlockSpec((_BN, _H), lambda i: (i, 0)),
        ],
        out_shape=[jax.ShapeDtypeStruct((_N, _H), _f32)] * 2,
    )(x, wl, wr, b)


def _combine(p0_r, p1_r, c0_r, c1_r, yr_r, relu):
    cnts = (c0_r[...] + c1_r[...]).reshape(_BN, _H)[:, 0:1]
    inv = 1.0 / jnp.maximum(cnts, 1.0)
    h = (p0_r[...] + p1_r[...]).reshape(_BN, _H) * inv + yr_r[...]
    if relu:
        h = jnp.maximum(h, 0.0)
    return h


def _p_specs():
    return [
        pl.BlockSpec((1, _BN, _H), lambda i: (0, i, 0)),
        pl.BlockSpec((1, _BN, _H), lambda i: (1, i, 0)),
        pl.BlockSpec((1, _BN, _H), lambda i: (0, i, 0)),
        pl.BlockSpec((1, _BN, _H), lambda i: (1, i, 0)),
    ]


def _tc_b_body(p0_r, p1_r, c0_r, c1_r, yrp_r, wl_r, wr_r, b_r, yl_r, yr_r):
    h = _combine(p0_r, p1_r, c0_r, c1_r, yrp_r, relu=True)
    yl_r[...] = jnp.dot(h, wl_r[...], preferred_element_type=_f32)
    yr_r[...] = jnp.dot(h, wr_r[...], preferred_element_type=_f32) + b_r[...]


def _tc_b(p, cn, yrp, wl, wr, b):
    return pl.pallas_call(
        _tc_b_body,
        grid=(_G,),
        in_specs=_p_specs() + [
            pl.BlockSpec((_BN, _H), lambda i: (i, 0)),
            pl.BlockSpec((_H, _H), lambda i: (0, 0)),
            pl.BlockSpec((_H, _H), lambda i: (0, 0)),
            pl.BlockSpec((1, _H), lambda i: (0, 0)),
        ],
        out_specs=[
            pl.BlockSpec((_BN, _H), lambda i: (i, 0)),
            pl.BlockSpec((_BN, _H), lambda i: (i, 0)),
        ],
        out_shape=[jax.ShapeDtypeStruct((_N, _H), _f32)] * 2,
    )(p, p, cn, cn, yrp, wl, wr, b)


def _tc_c_body(p0_r, p1_r, c0_r, c1_r, yrp_r, batch_r, wlin_r, blin_r,
               out_r, gsum, gcnt):
    i = pl.program_id(0)

    @pl.when(i == 0)
    def _():
        gsum[...] = jnp.zeros((_B, _H), _f32)
        gcnt[...] = jnp.zeros((_B, _H), _f32)

    h = _combine(p0_r, p1_r, c0_r, c1_r, yrp_r, relu=False)
    gids = batch_r[...].reshape(1, _BN)
    oh_t = (lax.broadcasted_iota(jnp.int32, (_B, 1), 0) == gids
            ).astype(_f32)
    gsum[...] += jnp.dot(oh_t, h, preferred_element_type=_f32)
    gcnt[...] += jnp.dot(oh_t, jnp.ones((_BN, _H), _f32),
                         preferred_element_type=_f32)

    @pl.when(i == _G - 1)
    def _():
        g = gsum[...] / jnp.maximum(gcnt[...], 1.0)
        out_r[...] = jnp.dot(g, wlin_r[...], preferred_element_type=_f32) \
            + blin_r[...]


def _tc_c(p, cn, yrp, batch3, wlin, blin):
    return pl.pallas_call(
        _tc_c_body,
        grid=(_G,),
        in_specs=_p_specs() + [
            pl.BlockSpec((_BN, _H), lambda i: (i, 0)),
            pl.BlockSpec((1, 1, _BN), lambda i: (i, 0, 0)),
            pl.BlockSpec((_H, _H), lambda i: (0, 0)),
            pl.BlockSpec((1, _H), lambda i: (0, 0)),
        ],
        out_specs=pl.BlockSpec((_B, _H), lambda i: (0, 0)),
        out_shape=jax.ShapeDtypeStruct((_B, _H), _f32),
        scratch_shapes=[
            pltpu.VMEM((_B, _H), _f32),
            pltpu.VMEM((_B, _H), _f32),
        ],
    )(p, p, cn, cn, yrp, batch3, wlin, blin)


def kernel(x, edge_index, batch, W1l, W1r, b1, W2l, W2r, b2, W3l, W3r, b3,
           Wlin, blin):
    src = edge_index[0]
    dst = edge_index[1]
    pad = _EPAD2 - _E
    lanes = jnp.arange(pad, dtype=jnp.int32) % _CHUNK
    srcp = jnp.concatenate([src, lanes]).reshape(_ROWS_ALLOC, _CHUNK)
    dstp = jnp.concatenate([dst, _N + lanes]).reshape(_ROWS_ALLOC, _CHUNK)

    z_slab = jnp.zeros((_SLAB, _H), _f32)
    ones_blk = jnp.ones((_CHUNK, _H), _f32)

    b1r = b1.reshape(1, _H)
    b2r = b2.reshape(1, _H)
    b3r = b3.reshape(1, _H)
    wlin_p = jnp.pad(Wlin, ((0, 0), (0, _H - _C)))
    blin_p = jnp.pad(blin, (0, _H - _C)).reshape(1, _H)
    batch3 = batch.reshape(_G, 1, _BN)

    agg = _make_sc_agg()
    cntk = _make_sc_cnt()

    yl1, yr1 = _tc_a(x, W1l, W1r, b1r)
    cn = cntk(dstp, z_slab, ones_blk)
    p = agg(srcp, dstp, yl1, z_slab)
    yl2, yr2 = _tc_b(p, cn, yr1, W2l, W2r, b2r)
    p = agg(srcp, dstp, yl2, z_slab)
    yl3, yr3 = _tc_b(p, cn, yr2, W3l, W3r, b3r)
    p = agg(srcp, dstp, yl3, z_slab)
    out = _tc_c(p, cn, yr3, batch3, wlin_p, blin_p)
    return out[:, :_C]


# speedup vs baseline: 2.5478x; 1.1448x over previous
import jax
import jax.numpy as jnp
from jax import lax
from jax.experimental import pallas as pl
from jax.experimental.pallas import tpu as pltpu
from jax.experimental.pallas import tpu_sc as plsc

_N = 10000
_E = 160000
_DIN = 1152
_H = 128
_B = 32
_C = 10

_NC = 2
_NS = 16
_CHUNK = 128
_TCH = 40
_EROWS = _NC * _NS * _TCH
_ROWS_ALLOC = _EROWS
_EPAD2 = _ROWS_ALLOC * _CHUNK
_SLAB = 640
_RPAD = _NS * _SLAB

_BN = 1000
_G = _N // _BN

_f32 = jnp.float32


def _sc_agg_body(src_h, dst_h, y_h, z_h, dep_h, p_h,
                 src_v, dst_v, b0, b1, acc, s0, s1):
    del dep_h
    c = lax.axis_index("c")
    s = lax.axis_index("s")
    slab = pl.ds(s * _SLAB, _SLAB)

    with jax.named_scope("agg_zero"):
        pltpu.sync_copy(z_h, acc.at[slab])

    with jax.named_scope("agg_stage"):
        base = (c * _NS + s) * _TCH
        pltpu.sync_copy(src_h.at[pl.ds(base, _TCH)], src_v)
        pltpu.sync_copy(dst_h.at[pl.ds(base, _TCH)], dst_v)

    plsc.subcore_barrier()

    bufs = (b0, b1)
    sems = (s0, s1)

    def fire(j, k):
        pltpu.async_copy(y_h.at[src_v.at[j]], bufs[k], sems[k])

    def wait(k):
        pltpu.make_async_copy(y_h.at[src_v.at[0]], bufs[k], sems[k]).wait()

    def scat(j, k):
        pltpu.sync_copy(bufs[k], acc.at[dst_v.at[j]], add=True)

    with jax.named_scope("agg_edges"):
        fire(0, 0)
        fire(1, 1)

        def pair(i, carry):
            j = 2 * i
            wait(0)
            scat(j, 0)
            fire(j + 2, 0)
            wait(1)
            scat(j + 1, 1)
            fire(j + 3, 1)
            return carry

        lax.fori_loop(0, (_TCH - 2) // 2, pair, 0)

        j = _TCH - 2
        wait(0)
        scat(j, 0)
        wait(1)
        scat(j + 1, 1)

    plsc.subcore_barrier()

    with jax.named_scope("agg_wb"):
        pltpu.sync_copy(acc.at[slab], p_h.at[c, slab])


def _make_sc_agg():
    mesh = plsc.VectorSubcoreMesh(
        core_axis_name="c", subcore_axis_name="s",
        num_cores=_NC, num_subcores=_NS)
    out_type = jax.ShapeDtypeStruct((_NC, _RPAD, _H), _f32)
    scratch = [
        pltpu.VMEM((_TCH, _CHUNK), jnp.int32),
        pltpu.VMEM((_TCH, _CHUNK), jnp.int32),
        pltpu.VMEM((_CHUNK, _H), _f32),
        pltpu.VMEM((_CHUNK, _H), _f32),
        pltpu.VMEM_SHARED((_RPAD, _H), _f32),
        pltpu.SemaphoreType.DMA,
        pltpu.SemaphoreType.DMA,
    ]
    return pl.kernel(
        _sc_agg_body, out_type=out_type, mesh=mesh, scratch_types=scratch)


def _sc_cnt_body(dst_h, z_h, ones_h, c_h, dst_v, ones_v, cacc, sem):
    c = lax.axis_index("c")
    s = lax.axis_index("s")
    slab = pl.ds(s * _SLAB, _SLAB)

    pltpu.sync_copy(z_h, cacc.at[slab])
    pltpu.sync_copy(ones_h, ones_v)
    base = (c * _NS + s) * _TCH
    pltpu.sync_copy(dst_h.at[pl.ds(base, _TCH)], dst_v)

    plsc.subcore_barrier()

    def step(j, carry):
        pltpu.async_copy(ones_v, cacc.at[dst_v.at[j]], sem, add=True)
        return carry

    lax.fori_loop(0, _TCH, step, 0)

    def drain(j, carry):
        pltpu.make_async_copy(ones_v, cacc.at[dst_v.at[0]], sem).wait()
        return carry

    lax.fori_loop(0, _TCH, drain, 0)

    plsc.subcore_barrier()

    pltpu.sync_copy(cacc.at[slab], c_h.at[c, slab])


def _make_sc_cnt():
    mesh = plsc.VectorSubcoreMesh(
        core_axis_name="c", subcore_axis_name="s",
        num_cores=_NC, num_subcores=_NS)
    out_type = jax.ShapeDtypeStruct((_NC, _RPAD, _H), _f32)
    scratch = [
        pltpu.VMEM((_TCH, _CHUNK), jnp.int32),
        pltpu.VMEM((_CHUNK, _H), _f32),
        pltpu.VMEM_SHARED((_RPAD, _H), _f32),
        pltpu.SemaphoreType.DMA,
    ]
    return pl.kernel(
        _sc_cnt_body, out_type=out_type, mesh=mesh, scratch_types=scratch)


def _tc_a_body(x_r, wl_r, wr_r, b_r, yl_r, yr_r):
    xb = x_r[...]
    yl_r[...] = jnp.dot(xb, wl_r[...], preferred_element_type=_f32)
    yr_r[...] = jnp.dot(xb, wr_r[...], preferred_element_type=_f32) + b_r[...]


def _tc_a(x, wl, wr, b):
    return pl.pallas_call(
        _tc_a_body,
        grid=(_G,),
        in_specs=[
            pl.BlockSpec((_BN, _DIN), lambda i: (i, 0)),
            pl.BlockSpec((_DIN, _H), lambda i: (0, 0)),
            pl.BlockSpec((_DIN, _H), lambda i: (0, 0)),
            pl.BlockSpec((1, _H), lambda i: (0, 0)),
        ],
        out_specs=[
            pl.BlockSpec((_BN, _H), lambda i: (i, 0)),
            pl.BlockSpec((_BN, _H), lambda i: (i, 0)),
        ],
        out_shape=[jax.ShapeDtypeStruct((_N, _H), _f32)] * 2,
    )(x, wl, wr, b)


def _combine(p0_r, p1_r, c0_r, c1_r, yr_r, relu):
    cnts = (c0_r[...] + c1_r[...]).reshape(_BN, _H)[:, 0:1]
    inv = 1.0 / jnp.maximum(cnts, 1.0)
    h = (p0_r[...] + p1_r[...]).reshape(_BN, _H) * inv + yr_r[...]
    if relu:
        h = jnp.maximum(h, 0.0)
    return h


def _p_specs():
    return [
        pl.BlockSpec((1, _BN, _H), lambda i: (0, i, 0)),
        pl.BlockSpec((1, _BN, _H), lambda i: (1, i, 0)),
        pl.BlockSpec((1, _BN, _H), lambda i: (0, i, 0)),
        pl.BlockSpec((1, _BN, _H), lambda i: (1, i, 0)),
    ]


def _tc_b_body(p0_r, p1_r, c0_r, c1_r, yrp_r, wl_r, wr_r, b_r, yl_r, yr_r):
    h = _combine(p0_r, p1_r, c0_r, c1_r, yrp_r, relu=True)
    yl_r[...] = jnp.dot(h, wl_r[...], preferred_element_type=_f32)
    yr_r[...] = jnp.dot(h, wr_r[...], preferred_element_type=_f32) + b_r[...]


def _tc_b(p, cn, yrp, wl, wr, b):
    return pl.pallas_call(
        _tc_b_body,
        grid=(_G,),
        in_specs=_p_specs() + [
            pl.BlockSpec((_BN, _H), lambda i: (i, 0)),
            pl.BlockSpec((_H, _H), lambda i: (0, 0)),
            pl.BlockSpec((_H, _H), lambda i: (0, 0)),
            pl.BlockSpec((1, _H), lambda i: (0, 0)),
        ],
        out_specs=[
            pl.BlockSpec((_BN, _H), lambda i: (i, 0)),
            pl.BlockSpec((_BN, _H), lambda i: (i, 0)),
        ],
        out_shape=[jax.ShapeDtypeStruct((_N, _H), _f32)] * 2,
    )(p, p, cn, cn, yrp, wl, wr, b)


def _tc_c_body(p0_r, p1_r, c0_r, c1_r, yrp_r, batch_r, wlin_r, blin_r,
               out_r, gsum, gcnt):
    i = pl.program_id(0)

    @pl.when(i == 0)
    def _():
        gsum[...] = jnp.zeros((_B, _H), _f32)
        gcnt[...] = jnp.zeros((_B, _H), _f32)

    h = _combine(p0_r, p1_r, c0_r, c1_r, yrp_r, relu=False)
    gids = batch_r[...].reshape(1, _BN)
    oh_t = (lax.broadcasted_iota(jnp.int32, (_B, 1), 0) == gids
            ).astype(_f32)
    gsum[...] += jnp.dot(oh_t, h, preferred_element_type=_f32)
    gcnt[...] += jnp.dot(oh_t, jnp.ones((_BN, _H), _f32),
                         preferred_element_type=_f32)

    @pl.when(i == _G - 1)
    def _():
        g = gsum[...] / jnp.maximum(gcnt[...], 1.0)
        out_r[...] = jnp.dot(g, wlin_r[...], preferred_element_type=_f32) \
            + blin_r[...]


def _tc_c(p, cn, yrp, batch3, wlin, blin):
    return pl.pallas_call(
        _tc_c_body,
        grid=(_G,),
        in_specs=_p_specs() + [
            pl.BlockSpec((_BN, _H), lambda i: (i, 0)),
            pl.BlockSpec((1, 1, _BN), lambda i: (i, 0, 0)),
            pl.BlockSpec((_H, _H), lambda i: (0, 0)),
            pl.BlockSpec((1, _H), lambda i: (0, 0)),
        ],
        out_specs=pl.BlockSpec((_B, _H), lambda i: (0, 0)),
        out_shape=jax.ShapeDtypeStruct((_B, _H), _f32),
        scratch_shapes=[
            pltpu.VMEM((_B, _H), _f32),
            pltpu.VMEM((_B, _H), _f32),
        ],
    )(p, p, cn, cn, yrp, batch3, wlin, blin)


def kernel(x, edge_index, batch, W1l, W1r, b1, W2l, W2r, b2, W3l, W3r, b3,
           Wlin, blin):
    src = edge_index[0]
    dst = edge_index[1]
    pad = _EPAD2 - _E
    lanes = jnp.arange(pad, dtype=jnp.int32) % _CHUNK
    srcp = jnp.concatenate([src, lanes]).reshape(_ROWS_ALLOC, _CHUNK)
    dstp = jnp.concatenate([dst, _N + lanes]).reshape(_ROWS_ALLOC, _CHUNK)

    z_slab = jnp.zeros((_SLAB, _H), _f32)
    ones_blk = jnp.ones((_CHUNK, _H), _f32)

    b1r = b1.reshape(1, _H)
    b2r = b2.reshape(1, _H)
    b3r = b3.reshape(1, _H)
    wlin_p = jnp.pad(Wlin, ((0, 0), (0, _H - _C)))
    blin_p = jnp.pad(blin, (0, _H - _C)).reshape(1, _H)
    batch3 = batch.reshape(_G, 1, _BN)

    agg = _make_sc_agg()
    cntk = _make_sc_cnt()

    yl1, yr1 = _tc_a(x, W1l, W1r, b1r)
    cn = cntk(dstp, z_slab, ones_blk)
    p = agg(srcp, dstp, yl1, z_slab, cn)
    yl2, yr2 = _tc_b(p, cn, yr1, W2l, W2r, b2r)
    p = agg(srcp, dstp, yl2, z_slab, cn)
    yl3, yr3 = _tc_b(p, cn, yr2, W3l, W3r, b3r)
    p = agg(srcp, dstp, yl3, z_slab, cn)
    out = _tc_c(p, cn, yr3, batch3, wlin_p, blin_p)
    return out[:, :_C]


# speedup vs baseline: 2.5981x; 1.0198x over previous
import jax
import jax.numpy as jnp
from jax import lax
from jax.experimental import pallas as pl
from jax.experimental.pallas import tpu as pltpu
from jax.experimental.pallas import tpu_sc as plsc

_N = 10000
_E = 160000
_DIN = 1152
_H = 128
_B = 32
_C = 10

_NC = 2
_NS = 16
_CHUNK = 128
_TCH = 40
_EROWS = _NC * _NS * _TCH
_ROWS_ALLOC = _EROWS
_EPAD2 = _ROWS_ALLOC * _CHUNK
_SLAB = 640
_RPAD = _NS * _SLAB

_BN = 1000
_G = _N // _BN

_f32 = jnp.float32


def _sc_agg_body(src_h, dst_h, y_h, z_h, dep_h, p_h,
                 src_v, dst_v, b0, b1, acc, s0, s1):
    del dep_h
    c = lax.axis_index("c")
    s = lax.axis_index("s")
    slab = pl.ds(s * _SLAB, _SLAB)

    with jax.named_scope("agg_zero"):
        pltpu.sync_copy(z_h, b0)

        def zstep(r, carry):
            pltpu.sync_copy(b0, acc.at[pl.ds(s * _SLAB + r * _CHUNK, _CHUNK)])
            return carry

        lax.fori_loop(0, _SLAB // _CHUNK, zstep, 0)

    with jax.named_scope("agg_stage"):
        base = (c * _NS + s) * _TCH
        pltpu.sync_copy(src_h.at[pl.ds(base, _TCH)], src_v)
        pltpu.sync_copy(dst_h.at[pl.ds(base, _TCH)], dst_v)

    plsc.subcore_barrier()

    bufs = (b0, b1)
    sems = (s0, s1)

    def fire(j, k):
        pltpu.async_copy(y_h.at[src_v.at[j]], bufs[k], sems[k])

    def wait(k):
        pltpu.make_async_copy(y_h.at[src_v.at[0]], bufs[k], sems[k]).wait()

    def scat(j, k):
        pltpu.sync_copy(bufs[k], acc.at[dst_v.at[j]], add=True)

    with jax.named_scope("agg_edges"):
        fire(0, 0)
        fire(1, 1)

        def pair(i, carry):
            j = 2 * i
            wait(0)
            scat(j, 0)
            fire(j + 2, 0)
            wait(1)
            scat(j + 1, 1)
            fire(j + 3, 1)
            return carry

        lax.fori_loop(0, (_TCH - 2) // 2, pair, 0)

        j = _TCH - 2
        wait(0)
        scat(j, 0)
        wait(1)
        scat(j + 1, 1)

    plsc.subcore_barrier()

    with jax.named_scope("agg_wb"):
        pltpu.sync_copy(acc.at[slab], p_h.at[c, slab])


def _make_sc_agg():
    mesh = plsc.VectorSubcoreMesh(
        core_axis_name="c", subcore_axis_name="s",
        num_cores=_NC, num_subcores=_NS)
    out_type = jax.ShapeDtypeStruct((_NC, _RPAD, _H), _f32)
    scratch = [
        pltpu.VMEM((_TCH, _CHUNK), jnp.int32),
        pltpu.VMEM((_TCH, _CHUNK), jnp.int32),
        pltpu.VMEM((_CHUNK, _H), _f32),
        pltpu.VMEM((_CHUNK, _H), _f32),
        pltpu.VMEM_SHARED((_RPAD, _H), _f32),
        pltpu.SemaphoreType.DMA,
        pltpu.SemaphoreType.DMA,
    ]
    return pl.kernel(
        _sc_agg_body, out_type=out_type, mesh=mesh, scratch_types=scratch)


def _sc_cnt_body(dst_h, z_h, ones_h, c_h, dst_v, ones_v, cacc, sem):
    c = lax.axis_index("c")
    s = lax.axis_index("s")
    slab = pl.ds(s * _SLAB, _SLAB)

    pltpu.sync_copy(z_h, ones_v)

    def zstep(r, carry):
        pltpu.sync_copy(ones_v, cacc.at[pl.ds(s * _SLAB + r * _CHUNK, _CHUNK)])
        return carry

    lax.fori_loop(0, _SLAB // _CHUNK, zstep, 0)
    pltpu.sync_copy(ones_h, ones_v)
    base = (c * _NS + s) * _TCH
    pltpu.sync_copy(dst_h.at[pl.ds(base, _TCH)], dst_v)

    plsc.subcore_barrier()

    def step(j, carry):
        pltpu.async_copy(ones_v, cacc.at[dst_v.at[j]], sem, add=True)
        return carry

    lax.fori_loop(0, _TCH, step, 0)

    def drain(j, carry):
        pltpu.make_async_copy(ones_v, cacc.at[dst_v.at[0]], sem).wait()
        return carry

    lax.fori_loop(0, _TCH, drain, 0)

    plsc.subcore_barrier()

    pltpu.sync_copy(cacc.at[slab], c_h.at[c, slab])


def _make_sc_cnt():
    mesh = plsc.VectorSubcoreMesh(
        core_axis_name="c", subcore_axis_name="s",
        num_cores=_NC, num_subcores=_NS)
    out_type = jax.ShapeDtypeStruct((_NC, _RPAD, _H), _f32)
    scratch = [
        pltpu.VMEM((_TCH, _CHUNK), jnp.int32),
        pltpu.VMEM((_CHUNK, _H), _f32),
        pltpu.VMEM_SHARED((_RPAD, _H), _f32),
        pltpu.SemaphoreType.DMA,
    ]
    return pl.kernel(
        _sc_cnt_body, out_type=out_type, mesh=mesh, scratch_types=scratch)


def _tc_a_body(x_r, wl_r, wr_r, b_r, yl_r, yr_r):
    xb = x_r[...]
    yl_r[...] = jnp.dot(xb, wl_r[...], preferred_element_type=_f32)
    yr_r[...] = jnp.dot(xb, wr_r[...], preferred_element_type=_f32) + b_r[...]


def _tc_a(x, wl, wr, b):
    return pl.pallas_call(
        _tc_a_body,
        grid=(_G,),
        in_specs=[
            pl.BlockSpec((_BN, _DIN), lambda i: (i, 0)),
            pl.BlockSpec((_DIN, _H), lambda i: (0, 0)),
            pl.BlockSpec((_DIN, _H), lambda i: (0, 0)),
            pl.BlockSpec((1, _H), lambda i: (0, 0)),
        ],
        out_specs=[
            pl.BlockSpec((_BN, _H), lambda i: (i, 0)),
            pl.BlockSpec((_BN, _H), lambda i: (i, 0)),
        ],
        out_shape=[jax.ShapeDtypeStruct((_N, _H), _f32)] * 2,
    )(x, wl, wr, b)


def _combine(p0_r, p1_r, c0_r, c1_r, yr_r, relu):
    cnts = (c0_r[...] + c1_r[...]).reshape(_BN, _H)[:, 0:1]
    inv = 1.0 / jnp.maximum(cnts, 1.0)
    h = (p0_r[...] + p1_r[...]).reshape(_BN, _H) * inv + yr_r[...]
    if relu:
        h = jnp.maximum(h, 0.0)
    return h


def _p_specs():
    return [
        pl.BlockSpec((1, _BN, _H), lambda i: (0, i, 0)),
        pl.BlockSpec((1, _BN, _H), lambda i: (1, i, 0)),
        pl.BlockSpec((1, _BN, _H), lambda i: (0, i, 0)),
        pl.BlockSpec((1, _BN, _H), lambda i: (1, i, 0)),
    ]


def _tc_b_body(p0_r, p1_r, c0_r, c1_r, yrp_r, wl_r, wr_r, b_r, yl_r, yr_r):
    h = _combine(p0_r, p1_r, c0_r, c1_r, yrp_r, relu=True)
    yl_r[...] = jnp.dot(h, wl_r[...], preferred_element_type=_f32)
    yr_r[...] = jnp.dot(h, wr_r[...], preferred_element_type=_f32) + b_r[...]


def _tc_b(p, cn, yrp, wl, wr, b):
    return pl.pallas_call(
        _tc_b_body,
        grid=(_G,),
        in_specs=_p_specs() + [
            pl.BlockSpec((_BN, _H), lambda i: (i, 0)),
            pl.BlockSpec((_H, _H), lambda i: (0, 0)),
            pl.BlockSpec((_H, _H), lambda i: (0, 0)),
            pl.BlockSpec((1, _H), lambda i: (0, 0)),
        ],
        out_specs=[
            pl.BlockSpec((_BN, _H), lambda i: (i, 0)),
            pl.BlockSpec((_BN, _H), lambda i: (i, 0)),
        ],
        out_shape=[jax.ShapeDtypeStruct((_N, _H), _f32)] * 2,
    )(p, p, cn, cn, yrp, wl, wr, b)


def _tc_c_body(p0_r, p1_r, c0_r, c1_r, yrp_r, batch_r, wlin_r, blin_r,
               out_r, gsum, gcnt):
    i = pl.program_id(0)

    @pl.when(i == 0)
    def _():
        gsum[...] = jnp.zeros((_B, _H), _f32)
        gcnt[...] = jnp.zeros((_B, _H), _f32)

    h = _combine(p0_r, p1_r, c0_r, c1_r, yrp_r, relu=False)
    gids = batch_r[...].reshape(1, _BN)
    oh_t = (lax.broadcasted_iota(jnp.int32, (_B, 1), 0) == gids
            ).astype(_f32)
    gsum[...] += jnp.dot(oh_t, h, preferred_element_type=_f32)
    gcnt[...] += jnp.dot(oh_t, jnp.ones((_BN, _H), _f32),
                         preferred_element_type=_f32)

    @pl.when(i == _G - 1)
    def _():
        g = gsum[...] / jnp.maximum(gcnt[...], 1.0)
        out_r[...] = jnp.dot(g, wlin_r[...], preferred_element_type=_f32) \
            + blin_r[...]


def _tc_c(p, cn, yrp, batch3, wlin, blin):
    return pl.pallas_call(
        _tc_c_body,
        grid=(_G,),
        in_specs=_p_specs() + [
            pl.BlockSpec((_BN, _H), lambda i: (i, 0)),
            pl.BlockSpec((1, 1, _BN), lambda i: (i, 0, 0)),
            pl.BlockSpec((_H, _H), lambda i: (0, 0)),
            pl.BlockSpec((1, _H), lambda i: (0, 0)),
        ],
        out_specs=pl.BlockSpec((_B, _H), lambda i: (0, 0)),
        out_shape=jax.ShapeDtypeStruct((_B, _H), _f32),
        scratch_shapes=[
            pltpu.VMEM((_B, _H), _f32),
            pltpu.VMEM((_B, _H), _f32),
        ],
    )(p, p, cn, cn, yrp, batch3, wlin, blin)


def kernel(x, edge_index, batch, W1l, W1r, b1, W2l, W2r, b2, W3l, W3r, b3,
           Wlin, blin):
    src = edge_index[0]
    dst = edge_index[1]
    pad = _EPAD2 - _E
    lanes = jnp.arange(pad, dtype=jnp.int32) % _CHUNK
    srcp = jnp.concatenate([src, lanes]).reshape(_ROWS_ALLOC, _CHUNK)
    dstp = jnp.concatenate([dst, _N + lanes]).reshape(_ROWS_ALLOC, _CHUNK)

    z_slab = jnp.zeros((_CHUNK, _H), _f32)
    ones_blk = jnp.ones((_CHUNK, _H), _f32)

    b1r = b1.reshape(1, _H)
    b2r = b2.reshape(1, _H)
    b3r = b3.reshape(1, _H)
    wlin_p = jnp.pad(Wlin, ((0, 0), (0, _H - _C)))
    blin_p = jnp.pad(blin, (0, _H - _C)).reshape(1, _H)
    batch3 = batch.reshape(_G, 1, _BN)

    agg = _make_sc_agg()
    cntk = _make_sc_cnt()

    cn = cntk(dstp, z_slab, ones_blk)
    yl1, yr1 = _tc_a(x, W1l, W1r, b1r)
    p = agg(srcp, dstp, yl1, z_slab, cn)
    yl2, yr2 = _tc_b(p, cn, yr1, W2l, W2r, b2r)
    p = agg(srcp, dstp, yl2, z_slab, cn)
    yl3, yr3 = _tc_b(p, cn, yr2, W3l, W3r, b3r)
    p = agg(srcp, dstp, yl3, z_slab, cn)
    out = _tc_c(p, cn, yr3, batch3, wlin_p, blin_p)
    return out[:, :_C]
